# baseline (speedup 1.0000x reference)
import functools

import jax
import jax.numpy as jnp
import numpy as np
from jax import lax
from jax.experimental import pallas as pl
from jax.experimental.pallas import tpu as pltpu
from jax.experimental.pallas import tpu_sc as plsc

N_NODES = 10000
N_EDGES = 160000
D_HALF = 128
DW = D_HALF // 2
NC = 2
NS = 16
L = 16
LB = 32

E_PER_TILE = N_EDGES // NS
CHUNK = 80
E_PAD_TILE = 10240
N_CHUNKS = E_PAD_TILE // CHUNK
STAGE = 32
N_STAGE = N_CHUNKS // STAGE
NPAIR = STAGE // 2
ZBLK = 40

_SIG = np.empty(D_HALF, dtype=np.int32)
for _q in range(D_HALF):
    _g, _u = _q // LB * LB, _q % LB
    _SIG[_q] = _g + (_u // 2 if _u % 2 == 0 else LB // 2 + _u // 2)
SIGMA = np.concatenate([_SIG, D_HALF + _SIG])


def _mm_body(x_ref, w_ref, o_ref):
    o_ref[...] = jnp.dot(x_ref[...], w_ref[...],
                         preferred_element_type=jnp.float32
                         ).astype(jnp.bfloat16)


def _support_split(features, weight):
    grid = (2, 5)
    return pl.pallas_call(
        _mm_body,
        grid=grid,
        in_specs=[
            pl.BlockSpec((2000, 256), lambda j, i: (i, 0)),
            pl.BlockSpec((256, 128), lambda j, i: (0, j)),
        ],
        out_specs=pl.BlockSpec((2000, 128), lambda j, i: (j * 5 + i, 0)),
        out_shape=jax.ShapeDtypeStruct((2 * N_NODES, D_HALF), jnp.bfloat16),
    )(features, weight)


@functools.partial(
    pl.kernel,
    mesh=plsc.VectorSubcoreMesh(core_axis_name="c", subcore_axis_name="s"),
    compiler_params=pltpu.CompilerParams(use_tc_tiling_on_sc=False,
                                         needs_layout_passes=False),
    out_type=jax.ShapeDtypeStruct((N_NODES, NC * D_HALF), jnp.float32),
    scratch_types=[
        pltpu.VMEM((STAGE, CHUNK), jnp.int32),
        pltpu.VMEM((STAGE, CHUNK), jnp.int32),
        pltpu.VMEM((STAGE, CHUNK), jnp.float32),
        pltpu.VMEM((2, CHUNK, DW), jnp.int32),
        pltpu.VMEM((CHUNK, D_HALF), jnp.float32),
        pltpu.VMEM_SHARED((N_NODES, D_HALF), jnp.float32),
        pltpu.SemaphoreType.DMA,
        pltpu.SemaphoreType.DMA,
    ],
)
def _sc_aggregate(support_hbm, src_hbm, dst_hbm, w_hbm, out_hbm,
                  src_v, dst_v, w_v, gbuf, fbuf, acc, gsem, ssem):
    c = lax.axis_index("c")
    s = lax.axis_index("s")

    base = s * 640
    nblk = jnp.where(s == NS - 1, 400 // ZBLK, 640 // ZBLK)

    zeros16 = jnp.zeros((L,), jnp.float32)

    def _zfill(k, _):
        r = k // (D_HALF // L)
        v = k % (D_HALF // L)
        fbuf[r, pl.ds(v * L, L)] = zeros16
        return 0

    lax.fori_loop(0, ZBLK * (D_HALF // L), _zfill, 0)

    def _zero_copy(i, _):
        st = pl.multiple_of(base + i * ZBLK, 8)
        pltpu.sync_copy(fbuf.at[pl.ds(0, ZBLK)], acc.at[pl.ds(st, ZBLK)])
        return 0

    lax.fori_loop(0, nblk, _zero_copy, 0)
    plsc.subcore_barrier()

    def _scale(p, j):
        for g in range(CHUNK // L):
            wvec = w_v[j, pl.ds(g * L, L)]
            for e2 in range(L):
                e = g * L + e2
                wf = jnp.full((L,), wvec[e2], jnp.float32)
                for v in range(DW // L):
                    pi = gbuf[p, e, pl.ds(v * L, L)]
                    bf = plsc.bitcast(pi, jnp.bfloat16)
                    lo, hi = plsc.unpack(
                        bf, format=plsc.PackFormat.INTERLEAVED)
                    fbuf[e, pl.ds(v * LB, L)] = lo * wf
                    fbuf[e, pl.ds(v * LB + L, L)] = hi * wf

    def _stage_body(b, _):
        pltpu.sync_copy(src_hbm.at[c, s, b], src_v)
        pltpu.sync_copy(dst_hbm.at[s, b], dst_v)
        pltpu.sync_copy(w_hbm.at[s, b], w_v)

        pltpu.async_copy(support_hbm.at[src_v.at[0]], gbuf.at[0], gsem)

        def _pair_body(t, _):
            j0 = 2 * t
            j1 = j0 + 1

            pltpu.async_copy(support_hbm.at[src_v.at[j1]], gbuf.at[1], gsem)
            pltpu.make_async_copy(
                support_hbm.at[src_v.at[j0]], gbuf.at[0], gsem).wait()

            @pl.when(t >= 1)
            def _():
                pltpu.make_async_copy(
                    fbuf, acc.at[dst_v.at[j0 - 1]], ssem).wait()

            _scale(0, j0)
            pltpu.async_copy(fbuf, acc.at[dst_v.at[j0]], ssem, add=True)

            @pl.when(t + 1 < NPAIR)
            def _():
                pltpu.async_copy(
                    support_hbm.at[src_v.at[j0 + 2]], gbuf.at[0], gsem)

            pltpu.make_async_copy(
                support_hbm.at[src_v.at[j1]], gbuf.at[1], gsem).wait()
            pltpu.make_async_copy(
                fbuf, acc.at[dst_v.at[j0]], ssem).wait()
            _scale(1, j1)
            pltpu.async_copy(fbuf, acc.at[dst_v.at[j1]], ssem, add=True)
            return 0

        lax.fori_loop(0, NPAIR, _pair_body, 0)
        pltpu.make_async_copy(
            fbuf, acc.at[dst_v.at[STAGE - 1]], ssem).wait()
        return 0

    lax.fori_loop(0, N_STAGE, _stage_body, 0)
    plsc.subcore_barrier()

    def _out_copy(i, _):
        st = pl.multiple_of(base + i * ZBLK, 8)
        pltpu.sync_copy(acc.at[pl.ds(st, ZBLK)],
                        out_hbm.at[pl.ds(st, ZBLK),
                                   pl.ds(pl.multiple_of(c * D_HALF, 128),
                                         D_HALF)])
        return 0

    lax.fori_loop(0, nblk, _out_copy, 0)


def kernel(features, adj_edge_index, adj_edge_weight, weight):
    dst = adj_edge_index[0].astype(jnp.int32)
    src = adj_edge_index[1].astype(jnp.int32)
    support = _support_split(features, weight[:, SIGMA])
    packed = lax.bitcast_convert_type(
        support.reshape(2 * N_NODES, DW, 2), jnp.int32)
    pad = ((0, 0), (0, E_PAD_TILE - E_PER_TILE))
    src_p = jnp.pad(src.reshape(NS, E_PER_TILE), pad)
    dst_p = jnp.pad(dst.reshape(NS, E_PER_TILE), pad)
    w_p = jnp.pad(adj_edge_weight.reshape(NS, E_PER_TILE), pad)
    src2 = jnp.stack([src_p, src_p + N_NODES]).reshape(
        NC, NS, N_STAGE, STAGE, CHUNK)
    dst3 = dst_p.reshape(NS, N_STAGE, STAGE, CHUNK)
    w3 = w_p.reshape(NS, N_STAGE, STAGE, CHUNK)
    return _sc_aggregate(packed, src2, dst3, w3)


# speedup vs baseline: 1.1804x; 1.1804x over previous
import functools

import jax
import jax.numpy as jnp
from jax import lax
from jax.experimental import pallas as pl
from jax.experimental.pallas import tpu as pltpu
from jax.experimental.pallas import tpu_sc as plsc

N_NODES = 10000
N_EDGES = 160000
D_HALF = 128
NC = 2
NS = 16
L = 16

E_PER_TILE = N_EDGES // NS
CHUNK = 80
N_CHUNKS = E_PER_TILE // CHUNK
STAGE = 25
N_STAGE = N_CHUNKS // STAGE
RBLK = 80


def _mm_body(x_ref, w_ref, o_ref):
    o_ref[...] = jnp.dot(x_ref[...], w_ref[...],
                         preferred_element_type=jnp.float32)


def _support_split(features, weight):
    grid = (2, 10)
    return pl.pallas_call(
        _mm_body,
        grid=grid,
        in_specs=[
            pl.BlockSpec((1000, 256), lambda j, i: (i, 0)),
            pl.BlockSpec((256, 128), lambda j, i: (0, j)),
        ],
        out_specs=pl.BlockSpec((1000, 128), lambda j, i: (j * 10 + i, 0)),
        out_shape=jax.ShapeDtypeStruct((2 * N_NODES, D_HALF), jnp.float32),
    )(features, weight)


@functools.partial(
    pl.kernel,
    mesh=plsc.VectorSubcoreMesh(core_axis_name="c", subcore_axis_name="s"),
    out_type=jax.ShapeDtypeStruct((N_NODES, NC * D_HALF), jnp.float32),
    scratch_types=[
        pltpu.VMEM((STAGE, CHUNK), jnp.int32),
        pltpu.VMEM((STAGE, CHUNK), jnp.int32),
        pltpu.VMEM((STAGE, CHUNK), jnp.float32),
        pltpu.VMEM((CHUNK, D_HALF), jnp.float32),
        pltpu.VMEM_SHARED((N_NODES, D_HALF), jnp.float32),
        pltpu.SemaphoreType.DMA,
    ],
)
def _sc_aggregate(support_hbm, src_hbm, dst_hbm, w_hbm, out_hbm,
                  src_v, dst_v, w_v, gbuf, acc, sem):
    c = lax.axis_index("c")
    s = lax.axis_index("s")

    zeros16 = jnp.zeros((L,), jnp.float32)

    def _zero_body(k, _):
        r = k // (D_HALF // L)
        v = k % (D_HALF // L)
        gbuf[r, pl.ds(v * L, L)] = zeros16
        return 0

    lax.fori_loop(0, RBLK * (D_HALF // L), _zero_body, 0)
    base = s * (8 * RBLK)
    nblk = jnp.where(s == NS - 1, 5, 8)

    def _zero_copy(i, _):
        st = pl.multiple_of(base + i * RBLK, 8)
        pltpu.sync_copy(gbuf, acc.at[pl.ds(st, RBLK)])
        return 0

    lax.fori_loop(0, nblk, _zero_copy, 0)
    plsc.subcore_barrier()

    def _stage_body(b, _):
        pltpu.sync_copy(src_hbm.at[c, s, b], src_v)
        pltpu.sync_copy(dst_hbm.at[s, b], dst_v)
        pltpu.sync_copy(w_hbm.at[s, b], w_v)

        def _chunk_body(j, _):
            pltpu.async_copy(support_hbm.at[src_v.at[j]], gbuf, sem).wait()

            def _scale_group(g, _):
                wvec = w_v[j, pl.ds(g * L, L)]
                for e2 in range(L):
                    e = g * L + e2
                    wv = jnp.full((L,), wvec[e2], jnp.float32)
                    for v in range(D_HALF // L):
                        sl = pl.ds(v * L, L)
                        gbuf[e, sl] = gbuf[e, sl] * wv
                return 0

            lax.fori_loop(0, CHUNK // L, _scale_group, 0)
            pltpu.sync_copy(gbuf, acc.at[dst_v.at[j]], add=True)
            return 0

        lax.fori_loop(0, STAGE, _chunk_body, 0)
        return 0

    lax.fori_loop(0, N_STAGE, _stage_body, 0)
    plsc.subcore_barrier()

    def _out_copy(i, _):
        st = pl.multiple_of(base + i * RBLK, 8)
        pltpu.sync_copy(acc.at[pl.ds(st, RBLK)],
                        out_hbm.at[pl.ds(st, RBLK),
                                   pl.ds(pl.multiple_of(c * D_HALF, 128),
                                         D_HALF)])
        return 0

    lax.fori_loop(0, nblk, _out_copy, 0)


def kernel(features, adj_edge_index, adj_edge_weight, weight):
    dst = adj_edge_index[0].astype(jnp.int32)
    src = adj_edge_index[1].astype(jnp.int32)
    support = _support_split(features, weight)
    src2 = jnp.stack([src, src + N_NODES]).reshape(
        NC, NS, N_STAGE, STAGE, CHUNK)
    dst3 = dst.reshape(NS, N_STAGE, STAGE, CHUNK)
    w3 = adj_edge_weight.reshape(NS, N_STAGE, STAGE, CHUNK)
    return _sc_aggregate(support, src2, dst3, w3)
